# speedup vs baseline: 16.0059x; 16.0059x over previous
import functools

import jax
import jax.numpy as jnp
from jax import lax
from jax.experimental import pallas as pl
from jax.experimental.pallas import tpu as pltpu
from jax.experimental.pallas import tpu_sc as plsc

N = 10000
E = 320000
D = 128

NC = 2
NS = 16
NW = NC * NS
CHUNK = 128
NUM_CHUNKS = E // CHUNK
NPAD = 10240
ROWS_PT = NPAD // NS

BM = 1000
GRID = N // BM

_MESH = dict(core_axis_name="c", subcore_axis_name="s", num_cores=NC,
             num_subcores=NS)


def _worker_id():
    return lax.axis_index("s") * NC + lax.axis_index("c")


def _num_chunks(wid):
    return jnp.where(wid < NUM_CHUNKS % NW, NUM_CHUNKS // NW + 1,
                     NUM_CHUNKS // NW)


@functools.partial(
    pl.kernel,
    out_type=jax.ShapeDtypeStruct((NC, NPAD), jnp.float32),
    mesh=plsc.VectorSubcoreMesh(**_MESH),
    scratch_types=[
        pltpu.VMEM((CHUNK,), jnp.int32),
        pltpu.VMEM((CHUNK,), jnp.float32),
        pltpu.VMEM_SHARED((NPAD,), jnp.float32),
    ],
)
def _sc_degree(dst_hbm, zeros_hbm, out_hbm, idx_d, ones_v, acc_sh):
    c = lax.axis_index("c")
    s = lax.axis_index("s")
    wid = _worker_id()
    row0 = s * ROWS_PT
    for j in range(CHUNK // 16):
        ones_v[pl.ds(16 * j, 16)] = jnp.ones((16,), jnp.float32)
    pltpu.sync_copy(zeros_hbm.at[pl.ds(row0, ROWS_PT)],
                    acc_sh.at[pl.ds(row0, ROWS_PT)])
    plsc.subcore_barrier()

    def body(i, carry):
        base = (wid + i * NW) * CHUNK
        pltpu.sync_copy(dst_hbm.at[pl.ds(base, CHUNK)], idx_d)
        pltpu.sync_copy(ones_v, acc_sh.at[idx_d], add=True)
        return carry

    lax.fori_loop(0, _num_chunks(wid), body, 0)
    plsc.subcore_barrier()
    pltpu.sync_copy(acc_sh.at[pl.ds(row0, ROWS_PT)],
                    out_hbm.at[c, pl.ds(row0, ROWS_PT)])


@functools.partial(
    pl.kernel,
    out_type=jax.ShapeDtypeStruct((NC, NPAD, D), jnp.float32),
    mesh=plsc.VectorSubcoreMesh(**_MESH),
    scratch_types=[
        pltpu.VMEM((CHUNK,), jnp.int32),
        pltpu.VMEM((CHUNK,), jnp.int32),
        pltpu.VMEM((CHUNK, D), jnp.float32),
        pltpu.VMEM_SHARED((NPAD, D), jnp.float32),
        pltpu.SemaphoreType.DMA,
    ],
)
def _sc_agg_rows(z_hbm, src_hbm, dst_hbm, zeros_hbm, out_hbm,
                 idx_s, idx_d, rows, acc_sh, sem):
    c = lax.axis_index("c")
    s = lax.axis_index("s")
    wid = _worker_id()
    row0 = s * ROWS_PT
    pltpu.sync_copy(zeros_hbm.at[pl.ds(row0, ROWS_PT)],
                    acc_sh.at[pl.ds(row0, ROWS_PT)])
    plsc.subcore_barrier()

    def body(i, carry):
        base = (wid + i * NW) * CHUNK
        pltpu.sync_copy(src_hbm.at[pl.ds(base, CHUNK)], idx_s)
        pltpu.sync_copy(dst_hbm.at[pl.ds(base, CHUNK)], idx_d)
        pltpu.async_copy(z_hbm.at[idx_s], rows, sem).wait()
        pltpu.sync_copy(rows, acc_sh.at[idx_d], add=True)
        return carry

    lax.fori_loop(0, _num_chunks(wid), body, 0)
    plsc.subcore_barrier()
    pltpu.sync_copy(acc_sh.at[pl.ds(row0, ROWS_PT)],
                    out_hbm.at[c, pl.ds(row0, ROWS_PT)])


@functools.partial(
    pl.kernel,
    out_type=jax.ShapeDtypeStruct((NC, NPAD), jnp.float32),
    mesh=plsc.VectorSubcoreMesh(**_MESH),
    scratch_types=[
        pltpu.VMEM((CHUNK,), jnp.int32),
        pltpu.VMEM((CHUNK,), jnp.int32),
        pltpu.VMEM((CHUNK,), jnp.float32),
        pltpu.VMEM_SHARED((NPAD,), jnp.float32),
        pltpu.SemaphoreType.DMA,
    ],
)
def _sc_agg_scalar(z_hbm, src_hbm, dst_hbm, zeros_hbm, out_hbm,
                   idx_s, idx_d, vals, acc_sh, sem):
    c = lax.axis_index("c")
    s = lax.axis_index("s")
    wid = _worker_id()
    row0 = s * ROWS_PT
    pltpu.sync_copy(zeros_hbm.at[pl.ds(row0, ROWS_PT)],
                    acc_sh.at[pl.ds(row0, ROWS_PT)])
    plsc.subcore_barrier()

    def body(i, carry):
        base = (wid + i * NW) * CHUNK
        pltpu.sync_copy(src_hbm.at[pl.ds(base, CHUNK)], idx_s)
        pltpu.sync_copy(dst_hbm.at[pl.ds(base, CHUNK)], idx_d)
        pltpu.async_copy(z_hbm.at[idx_s], vals, sem).wait()
        pltpu.sync_copy(vals, acc_sh.at[idx_d], add=True)
        return carry

    lax.fori_loop(0, _num_chunks(wid), body, 0)
    plsc.subcore_barrier()
    pltpu.sync_copy(acc_sh.at[pl.ds(row0, ROWS_PT)],
                    out_hbm.at[c, pl.ds(row0, ROWS_PT)])


def _dinv_block(deg_ref):
    return lax.rsqrt(deg_ref[:, 0:1] + deg_ref[:, 1:2] + 1.0)


def _tc_first_body(x_ref, w_ref, deg_ref, out_ref):
    dinv = _dinv_block(deg_ref)
    h = jnp.dot(x_ref[...], w_ref[...], preferred_element_type=jnp.float32,
                precision=lax.Precision.HIGHEST)
    out_ref[...] = h * dinv


def _tc_mid_body(a_ref, z_ref, deg_ref, b_ref, w_ref, out_ref):
    dinv = _dinv_block(deg_ref)
    pre = (a_ref[0] + a_ref[1] + z_ref[...]) * dinv + b_ref[...]
    h = jnp.where(pre >= 0, pre, 0.01 * pre)
    out_ref[...] = jnp.dot(h, w_ref[...], preferred_element_type=jnp.float32,
                           precision=lax.Precision.HIGHEST) * dinv


_tc_first = pl.pallas_call(
    _tc_first_body,
    grid=(GRID,),
    in_specs=[
        pl.BlockSpec((BM, D), lambda i: (i, 0)),
        pl.BlockSpec((D, D), lambda i: (0, 0)),
        pl.BlockSpec((BM, NC), lambda i: (i, 0)),
    ],
    out_specs=pl.BlockSpec((BM, D), lambda i: (i, 0)),
    out_shape=jax.ShapeDtypeStruct((N, D), jnp.float32),
)


def _make_tc_mid(width):
    return pl.pallas_call(
        _tc_mid_body,
        grid=(GRID,),
        in_specs=[
            pl.BlockSpec((NC, BM, D), lambda i: (0, i, 0)),
            pl.BlockSpec((BM, D), lambda i: (i, 0)),
            pl.BlockSpec((BM, NC), lambda i: (i, 0)),
            pl.BlockSpec((1, D), lambda i: (0, 0)),
            pl.BlockSpec((D, width), lambda i: (0, 0)),
        ],
        out_specs=pl.BlockSpec((BM, width), lambda i: (i, 0)),
        out_shape=jax.ShapeDtypeStruct((N, width), jnp.float32),
    )


_tc_mid_wide = _make_tc_mid(D)
_tc_mid_last = _make_tc_mid(1)


@jax.jit
def kernel(x, edge_index, W0, b0, W1, b1, W2, b2):
    src = edge_index[0]
    dst = edge_index[1]
    zeros2d = jnp.zeros((NPAD, D), jnp.float32)
    zeros1d = jnp.zeros((NPAD,), jnp.float32)

    deg = _sc_degree(dst, zeros1d)
    degT = deg[:, :N].T

    z0 = _tc_first(x, W0, degT)
    a0 = _sc_agg_rows(z0, src, dst, zeros2d)
    z1 = _tc_mid_wide(a0, z0, degT, b0.reshape(1, D), W1)
    a1 = _sc_agg_rows(z1, src, dst, zeros2d)
    z2 = _tc_mid_last(a1, z1, degT, b1.reshape(1, D), W2)
    a2 = _sc_agg_scalar(z2[:, 0], src, dst, zeros1d)

    dinv = lax.rsqrt(degT[:, 0] + degT[:, 1] + 1.0)
    y = (a2[0, :N] + a2[1, :N] + z2[:, 0]) * dinv + b2[0]
    return y[:, None]


# speedup vs baseline: 38.4802x; 2.4041x over previous
import functools

import jax
import jax.numpy as jnp
from jax import lax
from jax.experimental import pallas as pl
from jax.experimental.pallas import tpu as pltpu
from jax.experimental.pallas import tpu_sc as plsc

N = 10000
E = 320000
D = 128

NC = 2
NS = 16
NW = NC * NS
CHUNK = 128
NUM_CHUNKS = E // CHUNK
NIB = 4
NPAD = 10240
ROWS_PT = NPAD // NS
SPAN = (E // NW // CHUNK) * CHUNK
XTRA = (E - SPAN * NW) // CHUNK
EPWM = SPAN + CHUNK

BM = 1000
GRID = N // BM

_MESH = dict(core_axis_name="c", subcore_axis_name="s", num_cores=NC,
             num_subcores=NS)


def _worker_id():
    return lax.axis_index("s") * NC + lax.axis_index("c")


def _num_chunks(wid):
    return jnp.where(wid < NUM_CHUNKS % NW, NUM_CHUNKS // NW + 1,
                     NUM_CHUNKS // NW)


@functools.partial(
    pl.kernel,
    out_type=jax.ShapeDtypeStruct((NC, NPAD, D), jnp.float32),
    mesh=plsc.VectorSubcoreMesh(**_MESH),
    compiler_params=pltpu.CompilerParams(needs_layout_passes=False),
    scratch_types=[
        pltpu.VMEM((NIB, 2, CHUNK), jnp.int32),
        pltpu.VMEM((2, CHUNK, D), jnp.float32),
        pltpu.VMEM_SHARED((NPAD, D), jnp.float32),
        pltpu.SemaphoreType.DMA((NIB,)),
        pltpu.SemaphoreType.DMA((2,)),
        pltpu.SemaphoreType.DMA,
    ],
)
def _sc_agg_rows(z_hbm, ei_hbm, zeros_hbm, out_hbm,
                 ib, rows, acc_sh, isem, gsem, zsem):
    c = lax.axis_index("c")
    s = lax.axis_index("s")
    wid = _worker_id()
    row0 = s * ROWS_PT
    nch = _num_chunks(wid)

    def idx_dma(k, slot):
        base = (wid + k * NW) * CHUNK
        return pltpu.make_async_copy(
            ei_hbm.at[:, pl.ds(base, CHUNK)], ib.at[slot], isem.at[slot])

    def gather_dma(slot, b):
        return pltpu.make_async_copy(
            z_hbm.at[ib.at[slot, 0]], rows.at[b], gsem.at[b])

    zero = pltpu.make_async_copy(zeros_hbm,
                                 acc_sh.at[pl.ds(row0, ROWS_PT)], zsem)
    zero.start()

    for k in range(NIB - 1):
        idx_dma(k, k).start()
    idx_dma(0, 0).wait()
    gather_dma(0, 0).start()

    zero.wait()
    plsc.subcore_barrier()

    def body(i, carry):
        b = lax.rem(i, 2)
        sl = lax.rem(i, NIB)

        @pl.when(i + NIB - 1 < nch)
        def _():
            idx_dma(i + NIB - 1, lax.rem(i + NIB - 1, NIB)).start()

        @pl.when(i + 1 < nch)
        def _():
            sl1 = lax.rem(i + 1, NIB)
            idx_dma(i + 1, sl1).wait()
            gather_dma(sl1, 1 - b).start()

        gather_dma(sl, b).wait()
        pltpu.sync_copy(rows.at[b], acc_sh.at[ib.at[sl, 1]], add=True)
        return carry

    lax.fori_loop(0, nch, body, 0)
    plsc.subcore_barrier()
    pltpu.sync_copy(acc_sh.at[pl.ds(row0, ROWS_PT)],
                    out_hbm.at[c, pl.ds(row0, ROWS_PT)])


def _stage_reduce_store(s, c, acc, red, stage_sh, out_hbm):
    row0 = s * ROWS_PT
    pltpu.sync_copy(acc, stage_sh.at[s])
    plsc.subcore_barrier()
    pltpu.sync_copy(stage_sh.at[:, pl.ds(row0, ROWS_PT)], red)

    def rbody(j, carry):
        col = pl.ds(j * 16, 16)
        v = red[0, col]
        for t in range(1, NS):
            v = v + red[t, col]
        acc[col] = v
        return carry

    lax.fori_loop(0, ROWS_PT // 16, rbody, 0)
    pltpu.sync_copy(acc.at[pl.ds(0, ROWS_PT)],
                    out_hbm.at[c, pl.ds(row0, ROWS_PT)])


def _zero_acc(acc):
    zeros16 = jnp.zeros((16,), jnp.float32)

    def zbody(j, carry):
        acc[pl.ds(j * 16, 16)] = zeros16
        return carry

    lax.fori_loop(0, NPAD // 16, zbody, 0)


def _load_edge_span(ei_hbm, ebuf, wid):
    pltpu.sync_copy(ei_hbm.at[:, pl.ds(wid * SPAN, SPAN)],
                    ebuf.at[:, pl.ds(0, SPAN)])

    @pl.when(wid < XTRA)
    def _():
        pltpu.sync_copy(ei_hbm.at[:, pl.ds(NW * SPAN + wid * CHUNK, CHUNK)],
                        ebuf.at[:, pl.ds(SPAN, CHUNK)])

    return jnp.where(wid < XTRA, EPWM // 16, SPAN // 16)


@functools.partial(
    pl.kernel,
    out_type=jax.ShapeDtypeStruct((NC, NPAD), jnp.float32),
    mesh=plsc.VectorSubcoreMesh(**_MESH),
    compiler_params=pltpu.CompilerParams(needs_layout_passes=False),
    scratch_types=[
        pltpu.VMEM((2, EPWM), jnp.int32),
        pltpu.VMEM((NPAD,), jnp.float32),
        pltpu.VMEM((NS, ROWS_PT), jnp.float32),
        pltpu.VMEM_SHARED((NS, NPAD), jnp.float32),
    ],
)
def _sc_degree(ei_hbm, out_hbm, ebuf, acc, red, stage_sh):
    c = lax.axis_index("c")
    s = lax.axis_index("s")
    wid = _worker_id()
    nv = _load_edge_span(ei_hbm, ebuf, wid)
    _zero_acc(acc)
    ones16 = jnp.ones((16,), jnp.float32)

    def body(j, carry):
        idxv = ebuf[1, pl.ds(j * 16, 16)]
        plsc.addupdate_scatter(acc, [idxv], ones16)
        return carry

    lax.fori_loop(0, nv, body, 0)
    _stage_reduce_store(s, c, acc, red, stage_sh, out_hbm)


@functools.partial(
    pl.kernel,
    out_type=jax.ShapeDtypeStruct((NC, NPAD), jnp.float32),
    mesh=plsc.VectorSubcoreMesh(**_MESH),
    compiler_params=pltpu.CompilerParams(needs_layout_passes=False),
    scratch_types=[
        pltpu.VMEM((2, EPWM), jnp.int32),
        pltpu.VMEM((N,), jnp.float32),
        pltpu.VMEM((NPAD,), jnp.float32),
        pltpu.VMEM((NS, ROWS_PT), jnp.float32),
        pltpu.VMEM_SHARED((NS, NPAD), jnp.float32),
    ],
)
def _sc_agg_scalar(z_hbm, ei_hbm, out_hbm,
                   ebuf, zbuf, acc, red, stage_sh):
    c = lax.axis_index("c")
    s = lax.axis_index("s")
    wid = _worker_id()
    nv = _load_edge_span(ei_hbm, ebuf, wid)
    pltpu.sync_copy(z_hbm, zbuf)
    _zero_acc(acc)

    def body(j, carry):
        srcv = ebuf[0, pl.ds(j * 16, 16)]
        dstv = ebuf[1, pl.ds(j * 16, 16)]
        zv = plsc.load_gather(zbuf, [srcv])
        plsc.addupdate_scatter(acc, [dstv], zv)
        return carry

    lax.fori_loop(0, nv, body, 0)
    _stage_reduce_store(s, c, acc, red, stage_sh, out_hbm)


def _dinv_block(deg_ref):
    return lax.rsqrt(deg_ref[:, 0:1] + deg_ref[:, 1:2] + 1.0)


def _tc_first_body(x_ref, w_ref, deg_ref, out_ref):
    dinv = _dinv_block(deg_ref)
    h = jnp.dot(x_ref[...], w_ref[...], preferred_element_type=jnp.float32,
                precision=lax.Precision.HIGHEST)
    out_ref[...] = h * dinv


def _tc_mid_body(a_ref, z_ref, deg_ref, b_ref, w_ref, out_ref):
    dinv = _dinv_block(deg_ref)
    pre = (a_ref[0] + a_ref[1] + z_ref[...]) * dinv + b_ref[...]
    h = jnp.where(pre >= 0, pre, 0.01 * pre)
    out_ref[...] = jnp.dot(h, w_ref[...], preferred_element_type=jnp.float32,
                           precision=lax.Precision.HIGHEST) * dinv


_tc_first = pl.pallas_call(
    _tc_first_body,
    grid=(GRID,),
    in_specs=[
        pl.BlockSpec((BM, D), lambda i: (i, 0)),
        pl.BlockSpec((D, D), lambda i: (0, 0)),
        pl.BlockSpec((BM, NC), lambda i: (i, 0)),
    ],
    out_specs=pl.BlockSpec((BM, D), lambda i: (i, 0)),
    out_shape=jax.ShapeDtypeStruct((N, D), jnp.float32),
)


def _make_tc_mid(width):
    return pl.pallas_call(
        _tc_mid_body,
        grid=(GRID,),
        in_specs=[
            pl.BlockSpec((NC, BM, D), lambda i: (0, i, 0)),
            pl.BlockSpec((BM, D), lambda i: (i, 0)),
            pl.BlockSpec((BM, NC), lambda i: (i, 0)),
            pl.BlockSpec((1, D), lambda i: (0, 0)),
            pl.BlockSpec((D, width), lambda i: (0, 0)),
        ],
        out_specs=pl.BlockSpec((BM, width), lambda i: (i, 0)),
        out_shape=jax.ShapeDtypeStruct((N, width), jnp.float32),
    )


_tc_mid_wide = _make_tc_mid(D)
_tc_mid_last = _make_tc_mid(1)


@jax.jit
def kernel(x, edge_index, W0, b0, W1, b1, W2, b2):
    zeros2d = jnp.zeros((ROWS_PT, D), jnp.float32)

    deg = _sc_degree(edge_index)
    degT = deg[:, :N].T

    z0 = _tc_first(x, W0, degT)
    a0 = _sc_agg_rows(z0, edge_index, zeros2d)
    z1 = _tc_mid_wide(a0, z0, degT, b0.reshape(1, D), W1)
    a1 = _sc_agg_rows(z1, edge_index, zeros2d)
    z2 = _tc_mid_last(a1, z1, degT, b1.reshape(1, D), W2)
    a2 = _sc_agg_scalar(z2[:, 0], edge_index)

    dinv = lax.rsqrt(degT[:, 0] + degT[:, 1] + 1.0)
    y = (a2[0, :N] + a2[1, :N] + z2[:, 0]) * dinv + b2[0]
    return y[:, None]


# speedup vs baseline: 39.5217x; 1.0271x over previous
import functools

import jax
import jax.numpy as jnp
from jax import lax
from jax.experimental import pallas as pl
from jax.experimental.pallas import tpu as pltpu
from jax.experimental.pallas import tpu_sc as plsc

N = 10000
E = 320000
D = 128

NC = 2
NS = 16
NW = NC * NS
CHUNK = 128
NUM_CHUNKS = E // CHUNK
NIB = 4
NPAD = 10240
ROWS_PT = NPAD // NS
SPAN = (E // NW // CHUNK) * CHUNK
XTRA = (E - SPAN * NW) // CHUNK
EPWM = SPAN + CHUNK

BM = 1000
GRID = N // BM

_MESH = dict(core_axis_name="c", subcore_axis_name="s", num_cores=NC,
             num_subcores=NS)


def _worker_id():
    return lax.axis_index("s") * NC + lax.axis_index("c")


def _num_chunks(wid):
    return jnp.where(wid < NUM_CHUNKS % NW, NUM_CHUNKS // NW + 1,
                     NUM_CHUNKS // NW)


@functools.partial(
    pl.kernel,
    out_type=jax.ShapeDtypeStruct((NC, NPAD, D), jnp.float32),
    mesh=plsc.VectorSubcoreMesh(**_MESH),
    compiler_params=pltpu.CompilerParams(needs_layout_passes=False),
    scratch_types=[
        pltpu.VMEM((NIB, 2, CHUNK), jnp.int32),
        pltpu.VMEM((2, CHUNK, D), jnp.float32),
        pltpu.VMEM_SHARED((NPAD, D), jnp.float32),
        pltpu.SemaphoreType.DMA((NIB,)),
        pltpu.SemaphoreType.DMA((2,)),
        pltpu.SemaphoreType.DMA,
    ],
)
def _sc_agg_rows(z_hbm, ei_hbm, zeros_hbm, out_hbm,
                 ib, rows, acc_sh, isem, gsem, zsem):
    c = lax.axis_index("c")
    s = lax.axis_index("s")
    wid = _worker_id()
    row0 = s * ROWS_PT
    nch = _num_chunks(wid)

    def idx_dma(k, slot):
        base = (wid + k * NW) * CHUNK
        return pltpu.make_async_copy(
            ei_hbm.at[:, pl.ds(base, CHUNK)], ib.at[slot], isem.at[slot])

    def gather_dma(slot, b):
        return pltpu.make_async_copy(
            z_hbm.at[ib.at[slot, 0]], rows.at[b], gsem.at[b])

    zero = pltpu.make_async_copy(zeros_hbm,
                                 acc_sh.at[pl.ds(row0, ROWS_PT)], zsem)
    zero.start()

    for k in range(NIB - 1):
        idx_dma(k, k).start()
    idx_dma(0, 0).wait()
    gather_dma(0, 0).start()

    zero.wait()
    plsc.subcore_barrier()

    def body(i, carry):
        b = lax.rem(i, 2)
        sl = lax.rem(i, NIB)

        @pl.when(i + NIB - 1 < nch)
        def _():
            idx_dma(i + NIB - 1, lax.rem(i + NIB - 1, NIB)).start()

        @pl.when(i + 1 < nch)
        def _():
            sl1 = lax.rem(i + 1, NIB)
            idx_dma(i + 1, sl1).wait()
            gather_dma(sl1, 1 - b).start()

        gather_dma(sl, b).wait()
        pltpu.sync_copy(rows.at[b], acc_sh.at[ib.at[sl, 1]], add=True)
        return carry

    lax.fori_loop(0, nch, body, 0)
    plsc.subcore_barrier()
    pltpu.sync_copy(acc_sh.at[pl.ds(row0, ROWS_PT)],
                    out_hbm.at[c, pl.ds(row0, ROWS_PT)])


def _stage_reduce_store(s, c, acc, red, stage_sh, out_hbm):
    row0 = s * ROWS_PT
    pltpu.sync_copy(acc, stage_sh.at[s])
    plsc.subcore_barrier()
    pltpu.sync_copy(stage_sh.at[:, pl.ds(row0, ROWS_PT)], red)

    def rbody(j, carry):
        col = pl.ds(j * 16, 16)
        v = red[0, col]
        for t in range(1, NS):
            v = v + red[t, col]
        acc[col] = v
        return carry

    lax.fori_loop(0, ROWS_PT // 16, rbody, 0)
    pltpu.sync_copy(acc.at[pl.ds(0, ROWS_PT)],
                    out_hbm.at[c, pl.ds(row0, ROWS_PT)])


def _zero_acc(acc):
    zeros16 = jnp.zeros((16,), jnp.float32)

    def zbody(j, carry):
        acc[pl.ds(j * 16, 16)] = zeros16
        return carry

    lax.fori_loop(0, NPAD // 16, zbody, 0)


def _load_edge_span(ei_hbm, ebuf, wid):
    pltpu.sync_copy(ei_hbm.at[:, pl.ds(wid * SPAN, SPAN)],
                    ebuf.at[:, pl.ds(0, SPAN)])

    @pl.when(wid < XTRA)
    def _():
        pltpu.sync_copy(ei_hbm.at[:, pl.ds(NW * SPAN + wid * CHUNK, CHUNK)],
                        ebuf.at[:, pl.ds(SPAN, CHUNK)])

    return jnp.where(wid < XTRA, EPWM // 16, SPAN // 16)


@functools.partial(
    pl.kernel,
    out_type=jax.ShapeDtypeStruct((NC, NPAD), jnp.float32),
    mesh=plsc.VectorSubcoreMesh(**_MESH),
    compiler_params=pltpu.CompilerParams(needs_layout_passes=False),
    scratch_types=[
        pltpu.VMEM((2, EPWM), jnp.int32),
        pltpu.VMEM((NPAD,), jnp.float32),
        pltpu.VMEM((NS, ROWS_PT), jnp.float32),
        pltpu.VMEM_SHARED((NS, NPAD), jnp.float32),
    ],
)
def _sc_degree(ei_hbm, out_hbm, ebuf, acc, red, stage_sh):
    c = lax.axis_index("c")
    s = lax.axis_index("s")
    wid = _worker_id()
    nv = _load_edge_span(ei_hbm, ebuf, wid)
    _zero_acc(acc)
    ones16 = jnp.ones((16,), jnp.float32)

    def body(j, carry):
        idxv = ebuf[1, pl.ds(j * 16, 16)]
        plsc.addupdate_scatter(acc, [idxv], ones16)
        return carry

    lax.fori_loop(0, nv, body, 0)
    _stage_reduce_store(s, c, acc, red, stage_sh, out_hbm)


@functools.partial(
    pl.kernel,
    out_type=jax.ShapeDtypeStruct((NC, NPAD), jnp.float32),
    mesh=plsc.VectorSubcoreMesh(**_MESH),
    compiler_params=pltpu.CompilerParams(needs_layout_passes=False),
    scratch_types=[
        pltpu.VMEM((2, EPWM), jnp.int32),
        pltpu.VMEM((N,), jnp.float32),
        pltpu.VMEM((NPAD,), jnp.float32),
        pltpu.VMEM((NS, ROWS_PT), jnp.float32),
        pltpu.VMEM_SHARED((NS, NPAD), jnp.float32),
    ],
)
def _sc_agg_scalar(z_hbm, ei_hbm, out_hbm,
                   ebuf, zbuf, acc, red, stage_sh):
    c = lax.axis_index("c")
    s = lax.axis_index("s")
    wid = _worker_id()
    nv = _load_edge_span(ei_hbm, ebuf, wid)
    pltpu.sync_copy(z_hbm, zbuf)
    _zero_acc(acc)

    def body(j, carry):
        srcv = ebuf[0, pl.ds(j * 16, 16)]
        dstv = ebuf[1, pl.ds(j * 16, 16)]
        zv = plsc.load_gather(zbuf, [srcv])
        plsc.addupdate_scatter(acc, [dstv], zv)
        return carry

    lax.fori_loop(0, nv, body, 0)
    _stage_reduce_store(s, c, acc, red, stage_sh, out_hbm)


def _dinv_block(deg_ref):
    return lax.rsqrt(deg_ref[:, 0:1] + deg_ref[:, 1:2] + 1.0)


def _tc_first_body(x_ref, w_ref, deg_ref, out_ref):
    dinv = _dinv_block(deg_ref)
    h = jnp.dot(x_ref[...], w_ref[...], preferred_element_type=jnp.float32)
    out_ref[...] = h * dinv


def _tc_mid_body(a_ref, z_ref, deg_ref, b_ref, w_ref, out_ref):
    dinv = _dinv_block(deg_ref)
    pre = (a_ref[0] + a_ref[1] + z_ref[...]) * dinv + b_ref[...]
    h = jnp.where(pre >= 0, pre, 0.01 * pre)
    out_ref[...] = jnp.dot(h, w_ref[...],
                           preferred_element_type=jnp.float32) * dinv


_tc_first = pl.pallas_call(
    _tc_first_body,
    grid=(GRID,),
    in_specs=[
        pl.BlockSpec((BM, D), lambda i: (i, 0)),
        pl.BlockSpec((D, D), lambda i: (0, 0)),
        pl.BlockSpec((BM, NC), lambda i: (i, 0)),
    ],
    out_specs=pl.BlockSpec((BM, D), lambda i: (i, 0)),
    out_shape=jax.ShapeDtypeStruct((N, D), jnp.float32),
)


def _make_tc_mid(width):
    return pl.pallas_call(
        _tc_mid_body,
        grid=(GRID,),
        in_specs=[
            pl.BlockSpec((NC, BM, D), lambda i: (0, i, 0)),
            pl.BlockSpec((BM, D), lambda i: (i, 0)),
            pl.BlockSpec((BM, NC), lambda i: (i, 0)),
            pl.BlockSpec((1, D), lambda i: (0, 0)),
            pl.BlockSpec((D, width), lambda i: (0, 0)),
        ],
        out_specs=pl.BlockSpec((BM, width), lambda i: (i, 0)),
        out_shape=jax.ShapeDtypeStruct((N, width), jnp.float32),
    )


_tc_mid_wide = _make_tc_mid(D)
_tc_mid_last = _make_tc_mid(1)


@jax.jit
def kernel(x, edge_index, W0, b0, W1, b1, W2, b2):
    zeros2d = jnp.zeros((ROWS_PT, D), jnp.float32)

    deg = _sc_degree(edge_index)
    degT = deg[:, :N].T

    z0 = _tc_first(x, W0, degT)
    a0 = _sc_agg_rows(z0, edge_index, zeros2d)
    z1 = _tc_mid_wide(a0, z0, degT, b0.reshape(1, D), W1)
    a1 = _sc_agg_rows(z1, edge_index, zeros2d)
    z2 = _tc_mid_last(a1, z1, degT, b1.reshape(1, D), W2)
    a2 = _sc_agg_scalar(z2[:, 0], edge_index)

    dinv = lax.rsqrt(degT[:, 0] + degT[:, 1] + 1.0)
    y = (a2[0, :N] + a2[1, :N] + z2[:, 0]) * dinv + b2[0]
    return y[:, None]


# speedup vs baseline: 40.3587x; 1.0212x over previous
import functools

import jax
import jax.numpy as jnp
from jax import lax
from jax.experimental import pallas as pl
from jax.experimental.pallas import tpu as pltpu
from jax.experimental.pallas import tpu_sc as plsc

N = 10000
E = 320000
D = 128

NC = 2
NS = 16
NW = NC * NS
CHUNK = 128
NUM_CHUNKS = E // CHUNK
NIB = 4
NPAD = 10240
ROWS_PT = NPAD // NS
SPAN = (E // NW // CHUNK) * CHUNK
XTRA = (E - SPAN * NW) // CHUNK
EPWM = SPAN + CHUNK

BM = 2000
GRID = N // BM

_MESH = dict(core_axis_name="c", subcore_axis_name="s", num_cores=NC,
             num_subcores=NS)


def _worker_id():
    return lax.axis_index("s") * NC + lax.axis_index("c")


def _num_chunks(wid):
    return jnp.where(wid < NUM_CHUNKS % NW, NUM_CHUNKS // NW + 1,
                     NUM_CHUNKS // NW)


@functools.partial(
    pl.kernel,
    out_type=jax.ShapeDtypeStruct((NC, NPAD, D), jnp.float32),
    mesh=plsc.VectorSubcoreMesh(**_MESH),
    compiler_params=pltpu.CompilerParams(needs_layout_passes=False),
    scratch_types=[
        pltpu.VMEM((NIB, 2, CHUNK), jnp.int32),
        pltpu.VMEM((2, CHUNK, D), jnp.float32),
        pltpu.VMEM_SHARED((NPAD, D), jnp.float32),
        pltpu.SemaphoreType.DMA((NIB,)),
        pltpu.SemaphoreType.DMA((2,)),
        pltpu.SemaphoreType.DMA,
    ],
)
def _sc_agg_rows(z_hbm, ei_hbm, zeros_hbm, out_hbm,
                 ib, rows, acc_sh, isem, gsem, zsem):
    c = lax.axis_index("c")
    s = lax.axis_index("s")
    wid = _worker_id()
    row0 = s * ROWS_PT
    nch = _num_chunks(wid)

    def idx_dma(k, slot):
        base = (wid + k * NW) * CHUNK
        return pltpu.make_async_copy(
            ei_hbm.at[:, pl.ds(base, CHUNK)], ib.at[slot], isem.at[slot])

    def gather_dma(slot, b):
        return pltpu.make_async_copy(
            z_hbm.at[ib.at[slot, 0]], rows.at[b], gsem.at[b])

    zero = pltpu.make_async_copy(zeros_hbm,
                                 acc_sh.at[pl.ds(row0, ROWS_PT)], zsem)
    zero.start()

    for k in range(NIB - 1):
        idx_dma(k, k).start()
    idx_dma(0, 0).wait()
    gather_dma(0, 0).start()

    zero.wait()
    plsc.subcore_barrier()

    def body(i, carry):
        b = lax.rem(i, 2)
        sl = lax.rem(i, NIB)

        @pl.when(i + NIB - 1 < nch)
        def _():
            idx_dma(i + NIB - 1, lax.rem(i + NIB - 1, NIB)).start()

        @pl.when(i + 1 < nch)
        def _():
            sl1 = lax.rem(i + 1, NIB)
            idx_dma(i + 1, sl1).wait()
            gather_dma(sl1, 1 - b).start()

        gather_dma(sl, b).wait()
        pltpu.sync_copy(rows.at[b], acc_sh.at[ib.at[sl, 1]], add=True)
        return carry

    lax.fori_loop(0, nch, body, 0)
    plsc.subcore_barrier()
    pltpu.sync_copy(acc_sh.at[pl.ds(row0, ROWS_PT)],
                    out_hbm.at[c, pl.ds(row0, ROWS_PT)])


def _stage_reduce_store(s, c, acc, red, stage_sh, out_hbm):
    row0 = s * ROWS_PT
    pltpu.sync_copy(acc, stage_sh.at[s])
    plsc.subcore_barrier()
    pltpu.sync_copy(stage_sh.at[:, pl.ds(row0, ROWS_PT)], red)

    def rbody(j, carry):
        col = pl.ds(j * 16, 16)
        v = red[0, col]
        for t in range(1, NS):
            v = v + red[t, col]
        acc[col] = v
        return carry

    lax.fori_loop(0, ROWS_PT // 16, rbody, 0)
    pltpu.sync_copy(acc.at[pl.ds(0, ROWS_PT)],
                    out_hbm.at[c, pl.ds(row0, ROWS_PT)])


def _zero_acc(acc):
    zeros16 = jnp.zeros((16,), jnp.float32)

    def zbody(j, carry):
        acc[pl.ds(j * 16, 16)] = zeros16
        return carry

    lax.fori_loop(0, NPAD // 16, zbody, 0)


def _load_edge_span(ei_hbm, ebuf, wid):
    pltpu.sync_copy(ei_hbm.at[:, pl.ds(wid * SPAN, SPAN)],
                    ebuf.at[:, pl.ds(0, SPAN)])

    @pl.when(wid < XTRA)
    def _():
        pltpu.sync_copy(ei_hbm.at[:, pl.ds(NW * SPAN + wid * CHUNK, CHUNK)],
                        ebuf.at[:, pl.ds(SPAN, CHUNK)])

    return jnp.where(wid < XTRA, EPWM // 16, SPAN // 16)


@functools.partial(
    pl.kernel,
    out_type=jax.ShapeDtypeStruct((NC, NPAD), jnp.float32),
    mesh=plsc.VectorSubcoreMesh(**_MESH),
    compiler_params=pltpu.CompilerParams(needs_layout_passes=False),
    scratch_types=[
        pltpu.VMEM((2, EPWM), jnp.int32),
        pltpu.VMEM((NPAD,), jnp.float32),
        pltpu.VMEM((NS, ROWS_PT), jnp.float32),
        pltpu.VMEM_SHARED((NS, NPAD), jnp.float32),
    ],
)
def _sc_degree(ei_hbm, out_hbm, ebuf, acc, red, stage_sh):
    c = lax.axis_index("c")
    s = lax.axis_index("s")
    wid = _worker_id()
    nv = _load_edge_span(ei_hbm, ebuf, wid)
    _zero_acc(acc)
    ones16 = jnp.ones((16,), jnp.float32)

    def body(j, carry):
        idxv = ebuf[1, pl.ds(j * 16, 16)]
        plsc.addupdate_scatter(acc, [idxv], ones16)
        return carry

    lax.fori_loop(0, nv, body, 0)
    _stage_reduce_store(s, c, acc, red, stage_sh, out_hbm)


@functools.partial(
    pl.kernel,
    out_type=jax.ShapeDtypeStruct((NC, NPAD), jnp.float32),
    mesh=plsc.VectorSubcoreMesh(**_MESH),
    compiler_params=pltpu.CompilerParams(needs_layout_passes=False),
    scratch_types=[
        pltpu.VMEM((2, EPWM), jnp.int32),
        pltpu.VMEM((N,), jnp.float32),
        pltpu.VMEM((NPAD,), jnp.float32),
        pltpu.VMEM((NS, ROWS_PT), jnp.float32),
        pltpu.VMEM_SHARED((NS, NPAD), jnp.float32),
    ],
)
def _sc_agg_scalar(z_hbm, ei_hbm, out_hbm,
                   ebuf, zbuf, acc, red, stage_sh):
    c = lax.axis_index("c")
    s = lax.axis_index("s")
    wid = _worker_id()
    nv = _load_edge_span(ei_hbm, ebuf, wid)
    pltpu.sync_copy(z_hbm, zbuf)
    _zero_acc(acc)

    def body(j, carry):
        srcv = ebuf[0, pl.ds(j * 16, 16)]
        dstv = ebuf[1, pl.ds(j * 16, 16)]
        zv = plsc.load_gather(zbuf, [srcv])
        plsc.addupdate_scatter(acc, [dstv], zv)
        return carry

    lax.fori_loop(0, nv, body, 0)
    _stage_reduce_store(s, c, acc, red, stage_sh, out_hbm)


def _dinv_block(deg_ref):
    return lax.rsqrt(deg_ref[:, 0:1] + deg_ref[:, 1:2] + 1.0)


def _tc_first_body(x_ref, w_ref, deg_ref, out_ref):
    dinv = _dinv_block(deg_ref)
    h = jnp.dot(x_ref[...], w_ref[...], preferred_element_type=jnp.float32)
    out_ref[...] = h * dinv


def _tc_mid_body(a_ref, z_ref, deg_ref, b_ref, w_ref, out_ref):
    dinv = _dinv_block(deg_ref)
    pre = (a_ref[0] + a_ref[1] + z_ref[...]) * dinv + b_ref[...]
    h = jnp.where(pre >= 0, pre, 0.01 * pre)
    out_ref[...] = jnp.dot(h, w_ref[...],
                           preferred_element_type=jnp.float32) * dinv


_tc_first = pl.pallas_call(
    _tc_first_body,
    grid=(GRID,),
    in_specs=[
        pl.BlockSpec((BM, D), lambda i: (i, 0)),
        pl.BlockSpec((D, D), lambda i: (0, 0)),
        pl.BlockSpec((BM, NC), lambda i: (i, 0)),
    ],
    out_specs=pl.BlockSpec((BM, D), lambda i: (i, 0)),
    out_shape=jax.ShapeDtypeStruct((N, D), jnp.float32),
)


def _make_tc_mid(width):
    return pl.pallas_call(
        _tc_mid_body,
        grid=(GRID,),
        in_specs=[
            pl.BlockSpec((NC, BM, D), lambda i: (0, i, 0)),
            pl.BlockSpec((BM, D), lambda i: (i, 0)),
            pl.BlockSpec((BM, NC), lambda i: (i, 0)),
            pl.BlockSpec((1, D), lambda i: (0, 0)),
            pl.BlockSpec((D, width), lambda i: (0, 0)),
        ],
        out_specs=pl.BlockSpec((BM, width), lambda i: (i, 0)),
        out_shape=jax.ShapeDtypeStruct((N, width), jnp.float32),
    )


_tc_mid_wide = _make_tc_mid(D)
_tc_mid_last = _make_tc_mid(1)


@jax.jit
def kernel(x, edge_index, W0, b0, W1, b1, W2, b2):
    zeros2d = jnp.zeros((ROWS_PT, D), jnp.float32)

    deg = _sc_degree(edge_index)
    degT = deg[:, :N].T

    z0 = _tc_first(x, W0, degT)
    a0 = _sc_agg_rows(z0, edge_index, zeros2d)
    z1 = _tc_mid_wide(a0, z0, degT, b0.reshape(1, D), W1)
    a1 = _sc_agg_rows(z1, edge_index, zeros2d)
    z2 = _tc_mid_last(a1, z1, degT, b1.reshape(1, D), W2)
    a2 = _sc_agg_scalar(z2[:, 0], edge_index)

    dinv = lax.rsqrt(degT[:, 0] + degT[:, 1] + 1.0)
    y = (a2[0, :N] + a2[1, :N] + z2[:, 0]) * dinv + b2[0]
    return y[:, None]
